# Initial kernel scaffold; baseline (speedup 1.0000x reference)
import jax
import jax.numpy as jnp
from jax.experimental import pallas as pl


def kernel(h, edge_index, rel_ids, W):
    raise NotImplementedError("write your pallas kernel here")


# speedup vs baseline: 1.1475x; 1.1475x over previous
import functools

import jax
import jax.numpy as jnp
from jax import lax
from jax.experimental import pallas as pl
from jax.experimental.pallas import tpu as pltpu
from jax.experimental.pallas import tpu_sc as plsc

L = 16
NC = 2
NS = 16
NW = NC * NS
CHUNK = 80


def _dist_mult_body(D, EPW, NCHUNK, h_hbm, src_hbm, dst_hbm, rel_hbm, w_hbm,
                    out_hbm, sidx, didx, relv, ubuf, vbuf, wloc, obuf, sem):
    wid = lax.axis_index("s") * NC + lax.axis_index("c")
    pltpu.sync_copy(w_hbm, wloc)
    base0 = wid * EPW

    def chunk_body(c, carry):
        base = base0 + c * CHUNK
        pltpu.sync_copy(src_hbm.at[pl.ds(base, CHUNK)], sidx)
        pltpu.sync_copy(dst_hbm.at[pl.ds(base, CHUNK)], didx)
        pltpu.sync_copy(rel_hbm.at[pl.ds(base, CHUNK)], relv)
        cu = pltpu.async_copy(h_hbm.at[sidx], ubuf, sem)
        cv = pltpu.async_copy(h_hbm.at[didx], vbuf, sem)
        cu.wait()
        cv.wait()

        uflat = ubuf.reshape(1, CHUNK * D)
        vflat = vbuf.reshape(1, CHUNK * D)
        wflat = wloc.reshape(1, wloc.shape[0] * D)
        zero16 = jnp.zeros((L,), jnp.int32)

        def group_body(g, gcarry):
            rowbase = (jnp.full((L,), g * L, jnp.int32)
                       + lax.iota(jnp.int32, L)) * D
            relbase = relv[pl.ds(g * L, L)] * D
            acc = jnp.zeros((L,), jnp.float32)
            for d in range(D):
                iuv = rowbase + d
                u = plsc.load_gather(uflat, [zero16, iuv])
                v = plsc.load_gather(vflat, [zero16, iuv])
                r = plsc.load_gather(wflat, [zero16, relbase + d])
                acc = acc + u * v * r
            obuf[pl.ds(g * L, L)] = 1.0 / (1.0 + jnp.exp(-acc))
            return gcarry

        lax.fori_loop(0, CHUNK // L, group_body, 0)
        pltpu.sync_copy(obuf, out_hbm.at[pl.ds(base, CHUNK)])
        return carry

    lax.fori_loop(0, NCHUNK, chunk_body, 0)


def kernel(h, edge_index, rel_ids, W):
    E = edge_index.shape[1]
    D = h.shape[1]
    EPW = E // NW
    NCHUNK = EPW // CHUNK
    assert EPW * NW == E and NCHUNK * CHUNK == EPW

    src = edge_index[0]
    dst = edge_index[1]

    mesh = plsc.VectorSubcoreMesh(core_axis_name="c", subcore_axis_name="s")
    sc_kernel = functools.partial(
        pl.kernel,
        mesh=mesh,
        compiler_params=pltpu.CompilerParams(needs_layout_passes=False),
        out_type=jax.ShapeDtypeStruct((E,), jnp.float32),
        scratch_types=[
            pltpu.VMEM((CHUNK,), jnp.int32),
            pltpu.VMEM((CHUNK,), jnp.int32),
            pltpu.VMEM((CHUNK,), jnp.int32),
            pltpu.VMEM((CHUNK, D), jnp.float32),
            pltpu.VMEM((CHUNK, D), jnp.float32),
            pltpu.VMEM((W.shape[0], D), jnp.float32),
            pltpu.VMEM((CHUNK,), jnp.float32),
            pltpu.SemaphoreType.DMA,
        ],
    )(functools.partial(_dist_mult_body, D, EPW, NCHUNK))
    return sc_kernel(h, src, dst, rel_ids, W)


# speedup vs baseline: 1.2257x; 1.0682x over previous
import functools

import jax
import jax.numpy as jnp
from jax import lax
from jax.experimental import pallas as pl
from jax.experimental.pallas import tpu as pltpu
from jax.experimental.pallas import tpu_sc as plsc

L = 16
NC = 2
NS = 16
NW = NC * NS
CHUNK = 80
NACC = 4


def _dist_mult_body(D, NCHUNK, h_hbm, src_hbm, dst_hbm, rel_hbm, w_hbm,
                    out_hbm, sidx, didx, relv, ubuf, vbuf, wloc, obuf,
                    sem0, sem1):
    wid = lax.axis_index("s") * NC + lax.axis_index("c")
    pltpu.sync_copy(w_hbm, wloc)
    pltpu.sync_copy(src_hbm.at[wid], sidx)
    pltpu.sync_copy(dst_hbm.at[wid], didx)
    pltpu.sync_copy(rel_hbm.at[wid], relv)

    sems = (sem0, sem1)
    uflat = ubuf.reshape(1, 2 * CHUNK * D)
    vflat = vbuf.reshape(1, 2 * CHUNK * D)
    wflat = wloc.reshape(1, wloc.shape[0] * D)
    zero16 = jnp.zeros((L,), jnp.int32)
    iota16 = lax.iota(jnp.int32, L)

    def fire(c, b):
        cu = pltpu.make_async_copy(h_hbm.at[sidx.at[c]], ubuf.at[b], sems[b])
        cv = pltpu.make_async_copy(h_hbm.at[didx.at[c]], vbuf.at[b], sems[b])
        cu.start()
        cv.start()

    def wait(c, b):
        pltpu.make_async_copy(h_hbm.at[sidx.at[c]], ubuf.at[b], sems[b]).wait()
        pltpu.make_async_copy(h_hbm.at[didx.at[c]], vbuf.at[b], sems[b]).wait()

    def compute(c, b):
        wait(c, b)
        ubase = b * (CHUNK * D)

        def group_body(g, gcarry):
            rowbase = (jnp.full((L,), g * L, jnp.int32) + iota16) * D + ubase
            relbase = relv[c, pl.ds(g * L, L)] * D
            accs = [jnp.zeros((L,), jnp.float32) for _ in range(NACC)]
            for d in range(D):
                iuv = rowbase + d
                u = plsc.load_gather(uflat, [zero16, iuv])
                v = plsc.load_gather(vflat, [zero16, iuv])
                r = plsc.load_gather(wflat, [zero16, relbase + d])
                accs[d % NACC] = accs[d % NACC] + u * v * r
            acc = (accs[0] + accs[1]) + (accs[2] + accs[3])
            obuf[c, pl.ds(g * L, L)] = 1.0 / (1.0 + jnp.exp(-acc))
            return gcarry

        lax.fori_loop(0, CHUNK // L, group_body, 0)

    fire(0, 0)

    def superstep(s, carry):
        c0 = 2 * s
        fire(c0 + 1, 1)
        compute(c0, 0)
        fire(c0 + 2, 0)
        compute(c0 + 1, 1)
        return carry

    lax.fori_loop(0, (NCHUNK - 1) // 2, superstep, 0)
    compute(NCHUNK - 1, 0)

    pltpu.sync_copy(obuf, out_hbm.at[wid])


def kernel(h, edge_index, rel_ids, W):
    E = edge_index.shape[1]
    D = h.shape[1]
    EPW = E // NW
    NCHUNK = EPW // CHUNK
    assert EPW * NW == E and NCHUNK * CHUNK == EPW and NCHUNK % 2 == 1

    src = edge_index[0].reshape(NW, NCHUNK, CHUNK)
    dst = edge_index[1].reshape(NW, NCHUNK, CHUNK)
    rel = rel_ids.reshape(NW, NCHUNK, CHUNK)

    mesh = plsc.VectorSubcoreMesh(core_axis_name="c", subcore_axis_name="s")
    sc_kernel = functools.partial(
        pl.kernel,
        mesh=mesh,
        compiler_params=pltpu.CompilerParams(needs_layout_passes=False),
        out_type=jax.ShapeDtypeStruct((NW, NCHUNK, CHUNK), jnp.float32),
        scratch_types=[
            pltpu.VMEM((NCHUNK, CHUNK), jnp.int32),
            pltpu.VMEM((NCHUNK, CHUNK), jnp.int32),
            pltpu.VMEM((NCHUNK, CHUNK), jnp.int32),
            pltpu.VMEM((2, CHUNK, D), jnp.float32),
            pltpu.VMEM((2, CHUNK, D), jnp.float32),
            pltpu.VMEM((W.shape[0], D), jnp.float32),
            pltpu.VMEM((NCHUNK, CHUNK), jnp.float32),
            pltpu.SemaphoreType.DMA,
            pltpu.SemaphoreType.DMA,
        ],
    )(functools.partial(_dist_mult_body, D, NCHUNK))
    out = sc_kernel(h, src, dst, rel, W)
    return out.reshape(E)


# speedup vs baseline: 4.7118x; 3.8440x over previous
import functools

import jax
import jax.numpy as jnp
from jax import lax
from jax.experimental import pallas as pl
from jax.experimental.pallas import tpu as pltpu
from jax.experimental.pallas import tpu_sc as plsc

L = 16
NC = 2
NS = 16
NW = NC * NS
CHUNK = 80
NACC = 4


def _dist_mult_body(D, NCHUNK, h_hbm, src_hbm, dst_hbm, rel_hbm, w_hbm,
                    out_hbm, sidx, didx, relv, ubuf, vbuf, wloc, obuf,
                    sem0, sem1):
    wid = lax.axis_index("s") * NC + lax.axis_index("c")
    pltpu.sync_copy(w_hbm, wloc)
    pltpu.sync_copy(src_hbm.at[wid], sidx)
    pltpu.sync_copy(dst_hbm.at[wid], didx)
    pltpu.sync_copy(rel_hbm.at[wid], relv)

    sems = (sem0, sem1)
    iota16 = lax.iota(jnp.int32, L)

    def fire(c, b):
        cu = pltpu.make_async_copy(h_hbm.at[sidx.at[c]], ubuf.at[b], sems[b])
        cv = pltpu.make_async_copy(h_hbm.at[didx.at[c]], vbuf.at[b], sems[b])
        cu.start()
        cv.start()

    def wait(c, b):
        pltpu.make_async_copy(h_hbm.at[sidx.at[c]], ubuf.at[b], sems[b]).wait()
        pltpu.make_async_copy(h_hbm.at[didx.at[c]], vbuf.at[b], sems[b]).wait()

    def compute(c, b):
        wait(c, b)
        ub = ubuf.at[b]
        vb = vbuf.at[b]

        def group_body(g, gcarry):
            score = jnp.zeros((L,), jnp.float32)
            rel16 = relv[c, pl.ds(g * L, L)]
            for l in range(L):
                e = g * L + l
                rel_e = rel16[l]
                acc0 = jnp.zeros((L,), jnp.float32)
                acc1 = jnp.zeros((L,), jnp.float32)
                for j in range(D // L):
                    u = ub[e, pl.ds(j * L, L)]
                    v = vb[e, pl.ds(j * L, L)]
                    r = wloc[rel_e, pl.ds(j * L, L)]
                    if j % 2 == 0:
                        acc0 = acc0 + u * v * r
                    else:
                        acc1 = acc1 + u * v * r
                s = jnp.sum(acc0 + acc1)
                score = jnp.where(iota16 == l, s, score)
            obuf[c, pl.ds(g * L, L)] = 1.0 / (1.0 + jnp.exp(-score))
            return gcarry

        lax.fori_loop(0, CHUNK // L, group_body, 0)

    fire(0, 0)

    def superstep(s, carry):
        c0 = 2 * s
        fire(c0 + 1, 1)
        compute(c0, 0)
        fire(c0 + 2, 0)
        compute(c0 + 1, 1)
        return carry

    lax.fori_loop(0, (NCHUNK - 1) // 2, superstep, 0)
    compute(NCHUNK - 1, 0)

    pltpu.sync_copy(obuf, out_hbm.at[wid])


def kernel(h, edge_index, rel_ids, W):
    E = edge_index.shape[1]
    D = h.shape[1]
    EPW = E // NW
    NCHUNK = EPW // CHUNK
    assert EPW * NW == E and NCHUNK * CHUNK == EPW and NCHUNK % 2 == 1

    src = edge_index[0].reshape(NW, NCHUNK, CHUNK)
    dst = edge_index[1].reshape(NW, NCHUNK, CHUNK)
    rel = rel_ids.reshape(NW, NCHUNK, CHUNK)

    mesh = plsc.VectorSubcoreMesh(core_axis_name="c", subcore_axis_name="s")
    sc_kernel = functools.partial(
        pl.kernel,
        mesh=mesh,
        compiler_params=pltpu.CompilerParams(needs_layout_passes=False),
        out_type=jax.ShapeDtypeStruct((NW, NCHUNK, CHUNK), jnp.float32),
        scratch_types=[
            pltpu.VMEM((NCHUNK, CHUNK), jnp.int32),
            pltpu.VMEM((NCHUNK, CHUNK), jnp.int32),
            pltpu.VMEM((NCHUNK, CHUNK), jnp.int32),
            pltpu.VMEM((2, CHUNK, D), jnp.float32),
            pltpu.VMEM((2, CHUNK, D), jnp.float32),
            pltpu.VMEM((W.shape[0], D), jnp.float32),
            pltpu.VMEM((NCHUNK, CHUNK), jnp.float32),
            pltpu.SemaphoreType.DMA,
            pltpu.SemaphoreType.DMA,
        ],
    )(functools.partial(_dist_mult_body, D, NCHUNK))
    out = sc_kernel(h, src, dst, rel, W)
    return out.reshape(E)
